# baseline (speedup 1.0000x reference)
import functools

import jax
import jax.numpy as jnp
from jax import lax
from jax.experimental import pallas as pl
from jax.experimental.pallas import tpu as pltpu
from jax.experimental.pallas import tpu_sc as plsc

CHUNK = 128


def _sc_grid():
    info = plsc.get_sparse_core_info()
    return info.num_cores, info.num_subcores


def _edge_agg_kernel(nc, ns, n_pad, cpw, feat):
    nw = nc * ns
    rpt = n_pad // ns
    grp = 4
    mesh = plsc.VectorSubcoreMesh(core_axis_name="c", subcore_axis_name="s")

    @functools.partial(
        pl.kernel,
        mesh=mesh,
        compiler_params=pltpu.CompilerParams(use_tc_tiling_on_sc=False),
        out_type=jax.ShapeDtypeStruct((nc, n_pad, feat), jnp.float32),
        scratch_types=[
            pltpu.VMEM((cpw, CHUNK), jnp.int32),
            pltpu.VMEM((cpw, CHUNK), jnp.int32),
            pltpu.VMEM((2 * grp, CHUNK, feat), jnp.float32),
            pltpu.VMEM((CHUNK, feat), jnp.float32),
            pltpu.VMEM_SHARED((n_pad, feat), jnp.float32),
            pltpu.SemaphoreType.DMA,
            pltpu.SemaphoreType.DMA,
            pltpu.SemaphoreType.DMA,
            pltpu.SemaphoreType.DMA,
        ],
    )
    def agg(g_hbm, src_hbm, dst_hbm, zeros_hbm, out_hbm,
            srcbuf, dstbuf, rows, zbuf, acc, gs0, gs1, ss0, ss1):
        c = lax.axis_index("c")
        s = lax.axis_index("s")
        wid = s * nc + c
        pltpu.sync_copy(src_hbm.at[pl.ds(wid * cpw, cpw)], srcbuf)
        pltpu.sync_copy(dst_hbm.at[pl.ds(wid * cpw, cpw)], dstbuf)
        pltpu.sync_copy(zeros_hbm, zbuf)
        nfull, rem = rpt // CHUNK, rpt % CHUNK
        base = s * rpt

        @pl.loop(0, nfull)
        def _(i):
            pltpu.sync_copy(zbuf, acc.at[pl.ds(base + i * CHUNK, CHUNK)])

        if rem:
            pltpu.sync_copy(zbuf.at[pl.ds(0, rem)],
                            acc.at[pl.ds(base + nfull * CHUNK, rem)])
        plsc.subcore_barrier()

        gsem = (gs0, gs1)
        ssem = (ss0, ss1)
        nsuper = cpw // grp

        def gfire(j, slot, sem):
            pltpu.async_copy(g_hbm.at[srcbuf.at[j]], rows.at[slot], sem)

        def gdrain(j, slot, sem):
            pltpu.make_async_copy(g_hbm.at[srcbuf.at[j]], rows.at[slot],
                                  sem).wait()

        for b in range(grp):
            gfire(b, b, gs0)

        @pl.loop(0, nsuper // 2)
        def _(it):
            for p in (0, 1):
                sc_ = it * 2 + p
                j0 = sc_ * grp
                s0 = p * grp
                n0 = (1 - p) * grp
                for b in range(grp):
                    gdrain(j0 + b, s0 + b, gsem[p])
                for b in range(grp):
                    gfire(lax.rem(j0 + grp + b, cpw), n0 + b, gsem[1 - p])
                for b in range(grp):
                    pltpu.sync_copy(rows.at[s0 + b],
                                    acc.at[dstbuf.at[j0 + b]], add=True)

        for b in range(grp):
            gdrain(b, b, gs0)
        plsc.subcore_barrier()

        @pl.loop(0, nfull)
        def _(i):
            pltpu.sync_copy(acc.at[pl.ds(base + i * CHUNK, CHUNK)], rows.at[0])
            pltpu.sync_copy(rows.at[0],
                            out_hbm.at[c, pl.ds(base + i * CHUNK, CHUNK)])

        if rem:
            pltpu.sync_copy(acc.at[pl.ds(base + nfull * CHUNK, rem)],
                            rows.at[0, pl.ds(0, rem)])
            pltpu.sync_copy(rows.at[0, pl.ds(0, rem)],
                            out_hbm.at[c, pl.ds(base + nfull * CHUNK, rem)])

    return agg


def _deg_kernel(nc, ns, n_pad, cpw):
    rpt = n_pad // ns
    mesh = plsc.VectorSubcoreMesh(core_axis_name="c", subcore_axis_name="s")

    @functools.partial(
        pl.kernel,
        mesh=mesh,
        compiler_params=pltpu.CompilerParams(use_tc_tiling_on_sc=False),
        out_type=jax.ShapeDtypeStruct((nc, n_pad, 1), jnp.float32),
        scratch_types=[
            pltpu.VMEM((cpw, CHUNK), jnp.int32),
            pltpu.VMEM((CHUNK, 1), jnp.float32),
            pltpu.VMEM((CHUNK, 1), jnp.float32),
            pltpu.VMEM_SHARED((n_pad, 1), jnp.float32),
            pltpu.SemaphoreType.DMA,
        ],
    )
    def deg(dst_hbm, ones_hbm, zeros_hbm, out_hbm, dstbuf, onesbuf, zbuf, acc,
            dsem):
        c = lax.axis_index("c")
        s = lax.axis_index("s")
        wid = s * nc + c
        pltpu.sync_copy(dst_hbm.at[pl.ds(wid * cpw, cpw)], dstbuf)
        pltpu.sync_copy(ones_hbm, onesbuf)
        pltpu.sync_copy(zeros_hbm, zbuf)
        nfull, rem = rpt // CHUNK, rpt % CHUNK
        base = s * rpt

        @pl.loop(0, nfull)
        def _(i):
            pltpu.sync_copy(zbuf, acc.at[pl.ds(base + i * CHUNK, CHUNK)])

        if rem:
            pltpu.sync_copy(zbuf.at[pl.ds(0, rem)],
                            acc.at[pl.ds(base + nfull * CHUNK, rem)])
        plsc.subcore_barrier()

        @pl.loop(0, cpw)
        def _(j):
            pltpu.sync_copy(onesbuf, acc.at[dstbuf.at[j]], add=True)

        plsc.subcore_barrier()

        @pl.loop(0, nfull)
        def _(i):
            pltpu.sync_copy(acc.at[pl.ds(base + i * CHUNK, CHUNK)], zbuf)
            pltpu.sync_copy(zbuf,
                            out_hbm.at[c, pl.ds(base + i * CHUNK, CHUNK)])

        if rem:
            pltpu.sync_copy(acc.at[pl.ds(base + nfull * CHUNK, rem)],
                            zbuf.at[pl.ds(0, rem)])
            pltpu.sync_copy(zbuf.at[pl.ds(0, rem)],
                            out_hbm.at[c, pl.ds(base + nfull * CHUNK, rem)])

    return deg


def _mm1_body(x_ref, w_ref, degp_ref, g_ref, dinv_ref):
    p = degp_ref[...]
    deg = 1.0 + p[0] + p[1]
    dinv = lax.rsqrt(deg)
    h = jnp.dot(x_ref[...], w_ref[...], preferred_element_type=jnp.float32)
    g_ref[...] = h * dinv
    dinv_ref[...] = dinv


def _mm2_body(p_ref, g1_ref, dinv_ref, b1_ref, w2_ref, g2_ref):
    p = p_ref[...]
    dinv = dinv_ref[...]
    out1 = dinv * (p[0] + p[1] + g1_ref[...]) + b1_ref[...]
    z = jnp.maximum(out1, 0.0)
    h2 = jnp.dot(z, w2_ref[...], preferred_element_type=jnp.float32)
    g2_ref[...] = h2 * dinv


def _final_body(q_ref, g2_ref, dinv_ref, b2_ref, out_ref):
    q = q_ref[...]
    y = dinv_ref[...] * (q[0] + q[1] + g2_ref[...]) + b2_ref[...]
    m = jnp.max(y, axis=1, keepdims=True)
    e = jnp.exp(y - m)
    lse = m + jnp.log(jnp.sum(e, axis=1, keepdims=True))
    out_ref[...] = y - lse


def kernel(x, edge_index, W1, b1, W2, b2):
    n, d = x.shape
    h = W1.shape[1]
    cdim = W2.shape[1]
    e = edge_index.shape[1]
    nc, ns = _sc_grid()
    nw = nc * ns

    n_pad = ((n + 1 + 8 * ns - 1) // (8 * ns)) * (8 * ns)
    trash = n

    cpw = (e + nw * CHUNK - 1) // (nw * CHUNK)
    cpw = ((cpw + 7) // 8) * 8
    e_pad = nw * cpw * CHUNK

    src = edge_index[0].astype(jnp.int32)
    dst = edge_index[1].astype(jnp.int32)
    pad = e_pad - e
    src_p = jnp.concatenate([src, jnp.zeros((pad,), jnp.int32)])
    dst_p = jnp.concatenate([dst, jnp.full((pad,), trash, jnp.int32)])
    src_p = src_p.reshape(nw * cpw, CHUNK)
    dst_p = dst_p.reshape(nw * cpw, CHUNK)

    x_p = jnp.pad(x, ((0, n_pad - n), (0, 0)))
    ones_blk = jnp.ones((CHUNK, 1), jnp.float32)
    zeros_1 = jnp.zeros((CHUNK, 1), jnp.float32)
    zeros_h = jnp.zeros((CHUNK, h), jnp.float32)
    zeros_c = jnp.zeros((CHUNK, cdim), jnp.float32)
    b1r = b1.reshape(1, h)
    b2r = b2.reshape(1, cdim)

    degp = _deg_kernel(nc, ns, n_pad, cpw)(dst_p, ones_blk, zeros_1)

    blk = n_pad // 8
    grid = (n_pad // blk,)
    g1, dinv = pl.pallas_call(
        _mm1_body,
        grid=grid,
        in_specs=[
            pl.BlockSpec((blk, d), lambda i: (i, 0)),
            pl.BlockSpec((d, h), lambda i: (0, 0)),
            pl.BlockSpec((nc, blk, 1), lambda i: (0, i, 0)),
        ],
        out_specs=[
            pl.BlockSpec((blk, h), lambda i: (i, 0)),
            pl.BlockSpec((blk, 1), lambda i: (i, 0)),
        ],
        out_shape=[
            jax.ShapeDtypeStruct((n_pad, h), jnp.float32),
            jax.ShapeDtypeStruct((n_pad, 1), jnp.float32),
        ],
    )(x_p, W1, degp)

    p1 = _edge_agg_kernel(nc, ns, n_pad, cpw, h)(g1, src_p, dst_p, zeros_h)

    g2 = pl.pallas_call(
        _mm2_body,
        grid=grid,
        in_specs=[
            pl.BlockSpec((nc, blk, h), lambda i: (0, i, 0)),
            pl.BlockSpec((blk, h), lambda i: (i, 0)),
            pl.BlockSpec((blk, 1), lambda i: (i, 0)),
            pl.BlockSpec((1, h), lambda i: (0, 0)),
            pl.BlockSpec((h, cdim), lambda i: (0, 0)),
        ],
        out_specs=pl.BlockSpec((blk, cdim), lambda i: (i, 0)),
        out_shape=jax.ShapeDtypeStruct((n_pad, cdim), jnp.float32),
    )(p1, g1, dinv, b1r, W2)

    q = _edge_agg_kernel(nc, ns, n_pad, cpw, cdim)(g2, src_p, dst_p, zeros_c)

    out = pl.pallas_call(
        _final_body,
        grid=grid,
        in_specs=[
            pl.BlockSpec((nc, blk, cdim), lambda i: (0, i, 0)),
            pl.BlockSpec((blk, cdim), lambda i: (i, 0)),
            pl.BlockSpec((blk, 1), lambda i: (i, 0)),
            pl.BlockSpec((1, cdim), lambda i: (0, 0)),
        ],
        out_specs=pl.BlockSpec((blk, cdim), lambda i: (i, 0)),
        out_shape=jax.ShapeDtypeStruct((n_pad, cdim), jnp.float32),
    )(q, g2, dinv, b2r)

    return out[:n]


# speedup vs baseline: 1.1885x; 1.1885x over previous
import functools

import jax
import jax.numpy as jnp
from jax import lax
from jax.experimental import pallas as pl
from jax.experimental.pallas import tpu as pltpu
from jax.experimental.pallas import tpu_sc as plsc

CHUNK = 128


def _sc_grid():
    info = plsc.get_sparse_core_info()
    return info.num_cores, info.num_subcores


def _edge_agg_kernel(nc, ns, n_pad, cpw0, cpw1, feat):
    rpt = n_pad // ns
    cpm = max(cpw0, cpw1)
    c1base = ns * cpw0
    mesh = plsc.VectorSubcoreMesh(core_axis_name="c", subcore_axis_name="s")

    @functools.partial(
        pl.kernel,
        mesh=mesh,
        compiler_params=pltpu.CompilerParams(use_tc_tiling_on_sc=False),
        out_type=jax.ShapeDtypeStruct((nc, n_pad, feat), jnp.float32),
        scratch_types=[
            pltpu.VMEM((cpm, CHUNK), jnp.int32),
            pltpu.VMEM((cpm, CHUNK), jnp.int32),
            pltpu.VMEM((2, CHUNK, feat), jnp.float32),
            pltpu.VMEM((CHUNK, feat), jnp.float32),
            pltpu.VMEM_SHARED((n_pad, feat), jnp.float32),
            pltpu.SemaphoreType.DMA,
            pltpu.SemaphoreType.DMA,
        ],
    )
    def agg(g_hbm, src_hbm, dst_hbm, zeros_hbm, out_hbm,
            srcbuf, dstbuf, rows, zbuf, acc, gs0, gs1):
        c = lax.axis_index("c")
        s = lax.axis_index("s")
        mycpw = jnp.where(c == 0, cpw0, cpw1)

        @pl.when(c == 0)
        def _():
            pltpu.sync_copy(src_hbm.at[pl.ds(s * cpw0, cpw0)],
                            srcbuf.at[pl.ds(0, cpw0)])
            pltpu.sync_copy(dst_hbm.at[pl.ds(s * cpw0, cpw0)],
                            dstbuf.at[pl.ds(0, cpw0)])

        @pl.when(c == 1)
        def _():
            pltpu.sync_copy(src_hbm.at[pl.ds(c1base + s * cpw1, cpw1)],
                            srcbuf.at[pl.ds(0, cpw1)])
            pltpu.sync_copy(dst_hbm.at[pl.ds(c1base + s * cpw1, cpw1)],
                            dstbuf.at[pl.ds(0, cpw1)])

        pltpu.sync_copy(zeros_hbm, zbuf)
        nfull, rem = rpt // CHUNK, rpt % CHUNK
        base = s * rpt

        @pl.loop(0, nfull)
        def _(i):
            pltpu.sync_copy(zbuf, acc.at[pl.ds(base + i * CHUNK, CHUNK)])

        if rem:
            pltpu.sync_copy(zbuf.at[pl.ds(0, rem)],
                            acc.at[pl.ds(base + nfull * CHUNK, rem)])
        plsc.subcore_barrier()

        sems = (gs0, gs1)
        pltpu.async_copy(g_hbm.at[srcbuf.at[0]], rows.at[0], gs0)

        @pl.loop(0, mycpw // 2)
        def _(it):
            for b in (0, 1):
                j = it * 2 + b
                jn = lax.rem(j + 1, mycpw)
                pltpu.async_copy(g_hbm.at[srcbuf.at[jn]], rows.at[1 - b],
                                 sems[1 - b])
                pltpu.make_async_copy(g_hbm.at[srcbuf.at[j]], rows.at[b],
                                      sems[b]).wait()
                pltpu.sync_copy(rows.at[b], acc.at[dstbuf.at[j]], add=True)

        pltpu.make_async_copy(g_hbm.at[srcbuf.at[0]], rows.at[0], gs0).wait()
        plsc.subcore_barrier()

        @pl.loop(0, nfull)
        def _(i):
            pltpu.sync_copy(acc.at[pl.ds(base + i * CHUNK, CHUNK)], rows.at[0])
            pltpu.sync_copy(rows.at[0],
                            out_hbm.at[c, pl.ds(base + i * CHUNK, CHUNK)])

        if rem:
            pltpu.sync_copy(acc.at[pl.ds(base + nfull * CHUNK, rem)],
                            rows.at[0, pl.ds(0, rem)])
            pltpu.sync_copy(rows.at[0, pl.ds(0, rem)],
                            out_hbm.at[c, pl.ds(base + nfull * CHUNK, rem)])

    return agg


def _deg_kernel(nc, ns, n_pad, cpw0, cpw1):
    rpt = n_pad // ns
    cpm = max(cpw0, cpw1)
    c1base = ns * cpw0
    mesh = plsc.VectorSubcoreMesh(core_axis_name="c", subcore_axis_name="s")

    @functools.partial(
        pl.kernel,
        mesh=mesh,
        compiler_params=pltpu.CompilerParams(use_tc_tiling_on_sc=False),
        out_type=jax.ShapeDtypeStruct((nc, n_pad, 1), jnp.float32),
        scratch_types=[
            pltpu.VMEM((cpm, CHUNK), jnp.int32),
            pltpu.VMEM((CHUNK, 1), jnp.float32),
            pltpu.VMEM((CHUNK, 1), jnp.float32),
            pltpu.VMEM_SHARED((n_pad, 1), jnp.float32),
        ],
    )
    def deg(dst_hbm, ones_hbm, zeros_hbm, out_hbm, dstbuf, onesbuf, zbuf, acc):
        c = lax.axis_index("c")
        s = lax.axis_index("s")
        mycpw = jnp.where(c == 0, cpw0, cpw1)

        @pl.when(c == 0)
        def _():
            pltpu.sync_copy(dst_hbm.at[pl.ds(s * cpw0, cpw0)],
                            dstbuf.at[pl.ds(0, cpw0)])

        @pl.when(c == 1)
        def _():
            pltpu.sync_copy(dst_hbm.at[pl.ds(c1base + s * cpw1, cpw1)],
                            dstbuf.at[pl.ds(0, cpw1)])

        pltpu.sync_copy(ones_hbm, onesbuf)
        pltpu.sync_copy(zeros_hbm, zbuf)
        nfull, rem = rpt // CHUNK, rpt % CHUNK
        base = s * rpt

        @pl.loop(0, nfull)
        def _(i):
            pltpu.sync_copy(zbuf, acc.at[pl.ds(base + i * CHUNK, CHUNK)])

        if rem:
            pltpu.sync_copy(zbuf.at[pl.ds(0, rem)],
                            acc.at[pl.ds(base + nfull * CHUNK, rem)])
        plsc.subcore_barrier()

        @pl.loop(0, mycpw)
        def _(j):
            pltpu.sync_copy(onesbuf, acc.at[dstbuf.at[j]], add=True)

        plsc.subcore_barrier()

        @pl.loop(0, nfull)
        def _(i):
            pltpu.sync_copy(acc.at[pl.ds(base + i * CHUNK, CHUNK)], zbuf)
            pltpu.sync_copy(zbuf,
                            out_hbm.at[c, pl.ds(base + i * CHUNK, CHUNK)])

        if rem:
            pltpu.sync_copy(acc.at[pl.ds(base + nfull * CHUNK, rem)],
                            zbuf.at[pl.ds(0, rem)])
            pltpu.sync_copy(zbuf.at[pl.ds(0, rem)],
                            out_hbm.at[c, pl.ds(base + nfull * CHUNK, rem)])

    return deg


def _mm1_body(x_ref, w_ref, degp_ref, g_ref, dinv_ref):
    p = degp_ref[...]
    deg = 1.0 + p[0] + p[1]
    dinv = lax.rsqrt(deg)
    h = jnp.dot(x_ref[...], w_ref[...], preferred_element_type=jnp.float32)
    g_ref[...] = h * dinv
    dinv_ref[...] = dinv


def _mm2_body(p_ref, g1_ref, dinv_ref, b1_ref, w2_ref, g2_ref):
    p = p_ref[...]
    dinv = dinv_ref[...]
    out1 = dinv * (p[0] + p[1] + g1_ref[...]) + b1_ref[...]
    z = jnp.maximum(out1, 0.0)
    h2 = jnp.dot(z, w2_ref[...], preferred_element_type=jnp.float32)
    g2_ref[...] = h2 * dinv


def _final_body(q_ref, g2_ref, dinv_ref, b2_ref, out_ref):
    q = q_ref[...]
    y = dinv_ref[...] * (q[0] + q[1] + g2_ref[...]) + b2_ref[...]
    m = jnp.max(y, axis=1, keepdims=True)
    e = jnp.exp(y - m)
    lse = m + jnp.log(jnp.sum(e, axis=1, keepdims=True))
    out_ref[...] = y - lse


def kernel(x, edge_index, W1, b1, W2, b2):
    n, d = x.shape
    h = W1.shape[1]
    cdim = W2.shape[1]
    e = edge_index.shape[1]
    nc, ns = _sc_grid()
    nw = nc * ns

    n_pad = ((n + 1 + 8 * ns - 1) // (8 * ns)) * (8 * ns)
    trash = n

    frac0 = 0.35
    tch = (e + CHUNK - 1) // CHUNK
    cpw0 = max(2, int(round(tch * frac0 / ns / 2)) * 2)
    rem_ch = tch - ns * cpw0
    cpw1 = max(2, ((rem_ch + ns - 1) // ns + 1) // 2 * 2)
    tch_pad = ns * (cpw0 + cpw1)
    e_pad = tch_pad * CHUNK

    src = edge_index[0].astype(jnp.int32)
    dst = edge_index[1].astype(jnp.int32)
    pad = e_pad - e
    src_p = jnp.concatenate([src, jnp.zeros((pad,), jnp.int32)])
    dst_p = jnp.concatenate([dst, jnp.full((pad,), trash, jnp.int32)])
    src_p = src_p.reshape(tch_pad, CHUNK)
    dst_p = dst_p.reshape(tch_pad, CHUNK)

    x_p = jnp.pad(x, ((0, n_pad - n), (0, 0)))
    ones_blk = jnp.ones((CHUNK, 1), jnp.float32)
    zeros_1 = jnp.zeros((CHUNK, 1), jnp.float32)
    zeros_h = jnp.zeros((CHUNK, h), jnp.float32)
    zeros_c = jnp.zeros((CHUNK, cdim), jnp.float32)
    b1r = b1.reshape(1, h)
    b2r = b2.reshape(1, cdim)

    degp = _deg_kernel(nc, ns, n_pad, cpw0, cpw1)(dst_p, ones_blk, zeros_1)

    blk = n_pad // 8
    grid = (n_pad // blk,)
    g1, dinv = pl.pallas_call(
        _mm1_body,
        grid=grid,
        in_specs=[
            pl.BlockSpec((blk, d), lambda i: (i, 0)),
            pl.BlockSpec((d, h), lambda i: (0, 0)),
            pl.BlockSpec((nc, blk, 1), lambda i: (0, i, 0)),
        ],
        out_specs=[
            pl.BlockSpec((blk, h), lambda i: (i, 0)),
            pl.BlockSpec((blk, 1), lambda i: (i, 0)),
        ],
        out_shape=[
            jax.ShapeDtypeStruct((n_pad, h), jnp.float32),
            jax.ShapeDtypeStruct((n_pad, 1), jnp.float32),
        ],
    )(x_p, W1, degp)

    p1 = _edge_agg_kernel(nc, ns, n_pad, cpw0, cpw1, h)(
        g1, src_p, dst_p, zeros_h)

    g2 = pl.pallas_call(
        _mm2_body,
        grid=grid,
        in_specs=[
            pl.BlockSpec((nc, blk, h), lambda i: (0, i, 0)),
            pl.BlockSpec((blk, h), lambda i: (i, 0)),
            pl.BlockSpec((blk, 1), lambda i: (i, 0)),
            pl.BlockSpec((1, h), lambda i: (0, 0)),
            pl.BlockSpec((h, cdim), lambda i: (0, 0)),
        ],
        out_specs=pl.BlockSpec((blk, cdim), lambda i: (i, 0)),
        out_shape=jax.ShapeDtypeStruct((n_pad, cdim), jnp.float32),
    )(p1, g1, dinv, b1r, W2)

    q = _edge_agg_kernel(nc, ns, n_pad, cpw0, cpw1, cdim)(
        g2, src_p, dst_p, zeros_c)

    out = pl.pallas_call(
        _final_body,
        grid=grid,
        in_specs=[
            pl.BlockSpec((nc, blk, cdim), lambda i: (0, i, 0)),
            pl.BlockSpec((blk, cdim), lambda i: (i, 0)),
            pl.BlockSpec((blk, 1), lambda i: (i, 0)),
            pl.BlockSpec((1, cdim), lambda i: (0, 0)),
        ],
        out_specs=pl.BlockSpec((blk, cdim), lambda i: (i, 0)),
        out_shape=jax.ShapeDtypeStruct((n_pad, cdim), jnp.float32),
    )(q, g2, dinv, b2r)

    return out[:n]
